# Initial kernel scaffold; baseline (speedup 1.0000x reference)
import jax
import jax.numpy as jnp
from jax.experimental import pallas as pl


def kernel(entity_emb, user_emb, latent_emb, edge_index, edge_type, interact_indices, interact_values, relation_edge_weight, weight, disen_weight_att):
    raise NotImplementedError("write your pallas kernel here")


# speedup vs baseline: 1.7659x; 1.7659x over previous
import functools

import jax
import jax.numpy as jnp
from jax import lax
from jax.experimental import pallas as pl
from jax.experimental.pallas import tpu as pltpu
from jax.experimental.pallas import tpu_sc as plsc

_N_ENT = 10000
_N_USERS = 10000
_C = 128
_CH = 64
_N_REL = 16
_N_EDGES = 320000
_NNZ = 200000

_NS = 16
_CHUNK = 128
_E_CHUNKS = 160
_EP = _NS * _E_CHUNKS * _CHUNK
_U_CHUNKS = 100
_NP = _NS * _U_CHUNKS * _CHUNK
_ROWS = 10112
_STRIPE = _ROWS // _NS


def _sc_body(ent_cat, rel_cat, tail2, ridx2, head, cols2, vals, urows,
             z64, z16, o16, out_ent, out_usr,
             tail_buf, ridx_buf, head_buf, vals_buf, gbuf, rbuf,
             ones_buf, fin_buf, cnt_buf, acc, cnt_acc, sem):
    c = lax.axis_index("c")
    s = lax.axis_index("s")
    stripe = s * _STRIPE

    pltpu.sync_copy(z64, acc.at[pl.ds(stripe, _STRIPE)])
    pltpu.sync_copy(z16, cnt_acc.at[pl.ds(stripe, _STRIPE)])
    pltpu.sync_copy(o16, ones_buf)
    plsc.subcore_barrier()

    def ent_chunk(j, carry):
        off = (s * _E_CHUNKS + j) * _CHUNK
        pltpu.sync_copy(tail2.at[c, pl.ds(off, _CHUNK)], tail_buf)
        pltpu.sync_copy(ridx2.at[c, pl.ds(off, _CHUNK)], ridx_buf)
        pltpu.sync_copy(head.at[pl.ds(off, _CHUNK)], head_buf)
        pltpu.async_copy(ent_cat.at[tail_buf], gbuf, sem).wait()
        pltpu.async_copy(rel_cat.at[ridx_buf], rbuf, sem).wait()

        def mul_edge(e, cc):
            for g in range(_CH // 16):
                sl = pl.ds(g * 16, 16)
                gbuf[e, sl] = gbuf[e, sl] * rbuf[e, sl]
            return cc
        lax.fori_loop(0, _CHUNK, mul_edge, 0)

        pltpu.sync_copy(gbuf, acc.at[head_buf], add=True)
        pltpu.sync_copy(ones_buf, cnt_acc.at[head_buf], add=True)
        return carry
    lax.fori_loop(0, _E_CHUNKS, ent_chunk, 0)

    plsc.subcore_barrier()

    pltpu.sync_copy(acc.at[pl.ds(stripe, _STRIPE)], fin_buf)
    pltpu.sync_copy(cnt_acc.at[pl.ds(stripe, _STRIPE)], cnt_buf)

    def fin_row(r, carry):
        cv = jnp.maximum(cnt_buf[r, pl.ds(0, 16)], 1.0)
        for g in range(_CH // 16):
            sl = pl.ds(g * 16, 16)
            fin_buf[r, sl] = fin_buf[r, sl] / cv
        return carry
    lax.fori_loop(0, _STRIPE, fin_row, 0)
    pltpu.sync_copy(fin_buf, out_ent.at[c, pl.ds(stripe, _STRIPE)])

    plsc.subcore_barrier()
    pltpu.sync_copy(z64, acc.at[pl.ds(stripe, _STRIPE)])
    plsc.subcore_barrier()

    def usr_chunk(j, carry):
        off = (s * _U_CHUNKS + j) * _CHUNK
        pltpu.sync_copy(cols2.at[c, pl.ds(off, _CHUNK)], tail_buf)
        pltpu.sync_copy(urows.at[pl.ds(off, _CHUNK)], head_buf)
        pltpu.sync_copy(vals.at[pl.ds(off, _CHUNK)], vals_buf)
        pltpu.async_copy(ent_cat.at[tail_buf], gbuf, sem).wait()

        def mul_edge(e, cc):
            vv = vals_buf[e, pl.ds(0, 16)]
            for g in range(_CH // 16):
                sl = pl.ds(g * 16, 16)
                gbuf[e, sl] = gbuf[e, sl] * vv
            return cc
        lax.fori_loop(0, _CHUNK, mul_edge, 0)

        pltpu.sync_copy(gbuf, acc.at[head_buf], add=True)
        return carry
    lax.fori_loop(0, _U_CHUNKS, usr_chunk, 0)

    plsc.subcore_barrier()

    pltpu.sync_copy(acc.at[pl.ds(stripe, _STRIPE)], fin_buf)
    pltpu.sync_copy(fin_buf, out_usr.at[c, pl.ds(stripe, _STRIPE)])


_sc_agg = functools.partial(
    pl.kernel,
    out_type=(
        jax.ShapeDtypeStruct((2, _ROWS, _CH), jnp.float32),
        jax.ShapeDtypeStruct((2, _ROWS, _CH), jnp.float32),
    ),
    mesh=plsc.VectorSubcoreMesh(core_axis_name="c", subcore_axis_name="s"),
    scratch_types=[
        pltpu.VMEM((_CHUNK,), jnp.int32),
        pltpu.VMEM((_CHUNK,), jnp.int32),
        pltpu.VMEM((_CHUNK,), jnp.int32),
        pltpu.VMEM((_CHUNK, 16), jnp.float32),
        pltpu.VMEM((_CHUNK, _CH), jnp.float32),
        pltpu.VMEM((_CHUNK, _CH), jnp.float32),
        pltpu.VMEM((_CHUNK, 16), jnp.float32),
        pltpu.VMEM((_STRIPE, _CH), jnp.float32),
        pltpu.VMEM((_STRIPE, 16), jnp.float32),
        pltpu.VMEM_SHARED((_ROWS, _CH), jnp.float32),
        pltpu.VMEM_SHARED((_ROWS, 16), jnp.float32),
        pltpu.SemaphoreType.DMA,
    ],
    compiler_params=pltpu.CompilerParams(use_tc_tiling_on_sc=False),
)(_sc_body)


def _tc_body(u_ref, lat_ref, att_ref, w_ref, agg_ref, o_ref):
    att = att_ref[...]
    att = att - jnp.max(att, axis=-1, keepdims=True)
    att = jnp.exp(att)
    att = att / jnp.sum(att, axis=-1, keepdims=True)
    disen = lax.dot_general(att, w_ref[...], (((1,), (0,)), ((), ())),
                            preferred_element_type=jnp.float32)
    sc = lax.dot_general(u_ref[...], lat_ref[...], (((1,), (1,)), ((), ())),
                         preferred_element_type=jnp.float32)
    mask = jnp.where(lax.broadcasted_iota(jnp.int32, (1, 8), 1) >= 4,
                     -1e30, 0.0)
    sc = sc + mask
    sc = sc - jnp.max(sc, axis=-1, keepdims=True)
    sc = jnp.exp(sc)
    sc = sc / jnp.sum(sc, axis=-1, keepdims=True)
    coef = 1.0 + lax.dot_general(sc, disen, (((1,), (0,)), ((), ())),
                                 preferred_element_type=jnp.float32)
    o_ref[...] = agg_ref[...] * coef


def _tc_user(user_emb, latent_pad, att_pad, weight, usr_raw):
    blk = 1000
    grid = _N_USERS // blk
    return pl.pallas_call(
        _tc_body,
        grid=(grid,),
        in_specs=[
            pl.BlockSpec((blk, _C), lambda i: (i, 0)),
            pl.BlockSpec((8, _C), lambda i: (0, 0)),
            pl.BlockSpec((8, 8), lambda i: (0, 0)),
            pl.BlockSpec((8, _C), lambda i: (0, 0)),
            pl.BlockSpec((blk, _C), lambda i: (i, 0)),
        ],
        out_specs=pl.BlockSpec((blk, _C), lambda i: (i, 0)),
        out_shape=jax.ShapeDtypeStruct((_N_USERS, _C), jnp.float32),
    )(user_emb, latent_pad, att_pad, weight, usr_raw)


def kernel(entity_emb, user_emb, latent_emb, edge_index, edge_type,
           interact_indices, interact_values, relation_edge_weight,
           weight, disen_weight_att):
    i32 = jnp.int32
    head = edge_index[0]
    tail = edge_index[1]
    pad_e = _EP - _N_EDGES
    tail_pad = jnp.concatenate([tail, jnp.zeros((pad_e,), i32)])
    tail2 = jnp.stack([tail_pad, tail_pad + _N_ENT])
    tp = jnp.concatenate([edge_type - 1, jnp.full((pad_e,), _N_REL - 1, i32)])
    ridx2 = jnp.stack([tp, tp + _N_REL])
    head_pad = jnp.concatenate([head, jnp.full((pad_e,), _N_ENT, i32)])

    rows = interact_indices[0]
    cols = interact_indices[1]
    pad_n = _NP - _NNZ
    cols_pad = jnp.concatenate([cols, jnp.zeros((pad_n,), i32)])
    cols2 = jnp.stack([cols_pad, cols_pad + _N_ENT])
    vals_pad = jnp.concatenate([interact_values,
                                jnp.zeros((pad_n,), jnp.float32)])
    vals_wide = jnp.broadcast_to(vals_pad[:, None], (_NP, 16))
    urows_pad = jnp.concatenate([rows, jnp.full((pad_n,), _N_USERS, i32)])

    ent_cat = jnp.concatenate([entity_emb[:, :_CH], entity_emb[:, _CH:]],
                              axis=0)
    zrow = jnp.zeros((1, _CH), jnp.float32)
    rel_cat = jnp.concatenate(
        [relation_edge_weight[:, :_CH], zrow,
         relation_edge_weight[:, _CH:], zrow], axis=0)

    z64 = jnp.zeros((_STRIPE, _CH), jnp.float32)
    z16 = jnp.zeros((_STRIPE, 16), jnp.float32)
    o16 = jnp.ones((_CHUNK, 16), jnp.float32)

    out_ent, out_usr = _sc_agg(ent_cat, rel_cat, tail2, ridx2, head_pad,
                               cols2, vals_wide, urows_pad, z64, z16, o16)

    entity_agg = jnp.concatenate(
        [out_ent[0, :_N_ENT], out_ent[1, :_N_ENT]], axis=1)
    usr_raw = jnp.concatenate(
        [out_usr[0, :_N_USERS], out_usr[1, :_N_USERS]], axis=1)

    latent_pad = jnp.concatenate(
        [latent_emb, jnp.zeros((4, _C), jnp.float32)], axis=0)
    att_pad = jnp.concatenate(
        [disen_weight_att, jnp.zeros((4, 8), jnp.float32)], axis=0)

    user_agg = _tc_user(user_emb, latent_pad, att_pad, weight, usr_raw)
    return (entity_agg, user_agg)


# speedup vs baseline: 1.8226x; 1.0321x over previous
import functools

import jax
import jax.numpy as jnp
from jax import lax
from jax.experimental import pallas as pl
from jax.experimental.pallas import tpu as pltpu
from jax.experimental.pallas import tpu_sc as plsc

_N_ENT = 10000
_N_USERS = 10000
_C = 128
_CH = 64
_N_REL = 16
_N_EDGES = 320000
_NNZ = 200000

_NS = 16
_CHUNK = 128
_E_CHUNKS = 160
_EP = _NS * _E_CHUNKS * _CHUNK
_U_CHUNKS = 100
_NP = _NS * _U_CHUNKS * _CHUNK
_ROWS = 10112
_STRIPE = _ROWS // _NS


def _sc_body(ent_cat, rel_cat, tail2, ridx2, head, cols2, vals, urows,
             z64, z16, o16, out_ent, out_usr,
             tail_buf, ridx_buf, head_buf, vals_buf, gbuf, rbuf,
             ones_buf, fin_buf, cnt_buf, acc, cnt_acc, sem):
    c = lax.axis_index("c")
    s = lax.axis_index("s")
    stripe = s * _STRIPE

    pltpu.sync_copy(z64, acc.at[pl.ds(stripe, _STRIPE)])
    pltpu.sync_copy(z16, cnt_acc.at[pl.ds(stripe, _STRIPE)])
    pltpu.sync_copy(o16, ones_buf)
    plsc.subcore_barrier()

    def ent_chunk(j, carry):
        off = (s * _E_CHUNKS + j) * _CHUNK
        pltpu.sync_copy(tail2.at[c, pl.ds(off, _CHUNK)], tail_buf)
        pltpu.sync_copy(ridx2.at[c, pl.ds(off, _CHUNK)], ridx_buf)
        pltpu.sync_copy(head.at[pl.ds(off, _CHUNK)], head_buf)
        pltpu.async_copy(ent_cat.at[tail_buf], gbuf, sem).wait()
        pltpu.async_copy(rel_cat.at[ridx_buf], rbuf, sem).wait()

        @plsc.parallel_loop(0, _CHUNK, 1, unroll=8)
        def mul_edge(e):
            for g in range(_CH // 16):
                sl = pl.ds(g * 16, 16)
                gbuf[e, sl] = gbuf[e, sl] * rbuf[e, sl]

        pltpu.sync_copy(gbuf, acc.at[head_buf], add=True)
        pltpu.sync_copy(ones_buf, cnt_acc.at[head_buf], add=True)
        return carry
    lax.fori_loop(0, _E_CHUNKS, ent_chunk, 0)

    plsc.subcore_barrier()

    pltpu.sync_copy(acc.at[pl.ds(stripe, _STRIPE)], fin_buf)
    pltpu.sync_copy(cnt_acc.at[pl.ds(stripe, _STRIPE)], cnt_buf)

    @plsc.parallel_loop(0, _STRIPE, 1, unroll=4)
    def fin_row(r):
        cv = jnp.maximum(cnt_buf[r, pl.ds(0, 16)], 1.0)
        rcv = 1.0 / cv
        for g in range(_CH // 16):
            sl = pl.ds(g * 16, 16)
            fin_buf[r, sl] = fin_buf[r, sl] * rcv
    pltpu.sync_copy(fin_buf, out_ent.at[c, pl.ds(stripe, _STRIPE)])

    plsc.subcore_barrier()
    pltpu.sync_copy(z64, acc.at[pl.ds(stripe, _STRIPE)])
    plsc.subcore_barrier()

    def usr_chunk(j, carry):
        off = (s * _U_CHUNKS + j) * _CHUNK
        pltpu.sync_copy(cols2.at[c, pl.ds(off, _CHUNK)], tail_buf)
        pltpu.sync_copy(urows.at[pl.ds(off, _CHUNK)], head_buf)
        pltpu.sync_copy(vals.at[pl.ds(off, _CHUNK)], vals_buf)
        pltpu.async_copy(ent_cat.at[tail_buf], gbuf, sem).wait()

        @plsc.parallel_loop(0, _CHUNK, 1, unroll=8)
        def mul_edge(e):
            vv = vals_buf[e, pl.ds(0, 16)]
            for g in range(_CH // 16):
                sl = pl.ds(g * 16, 16)
                gbuf[e, sl] = gbuf[e, sl] * vv

        pltpu.sync_copy(gbuf, acc.at[head_buf], add=True)
        return carry
    lax.fori_loop(0, _U_CHUNKS, usr_chunk, 0)

    plsc.subcore_barrier()

    pltpu.sync_copy(acc.at[pl.ds(stripe, _STRIPE)], fin_buf)
    pltpu.sync_copy(fin_buf, out_usr.at[c, pl.ds(stripe, _STRIPE)])


_sc_agg = functools.partial(
    pl.kernel,
    out_type=(
        jax.ShapeDtypeStruct((2, _ROWS, _CH), jnp.float32),
        jax.ShapeDtypeStruct((2, _ROWS, _CH), jnp.float32),
    ),
    mesh=plsc.VectorSubcoreMesh(core_axis_name="c", subcore_axis_name="s"),
    scratch_types=[
        pltpu.VMEM((_CHUNK,), jnp.int32),
        pltpu.VMEM((_CHUNK,), jnp.int32),
        pltpu.VMEM((_CHUNK,), jnp.int32),
        pltpu.VMEM((_CHUNK, 16), jnp.float32),
        pltpu.VMEM((_CHUNK, _CH), jnp.float32),
        pltpu.VMEM((_CHUNK, _CH), jnp.float32),
        pltpu.VMEM((_CHUNK, 16), jnp.float32),
        pltpu.VMEM((_STRIPE, _CH), jnp.float32),
        pltpu.VMEM((_STRIPE, 16), jnp.float32),
        pltpu.VMEM_SHARED((_ROWS, _CH), jnp.float32),
        pltpu.VMEM_SHARED((_ROWS, 16), jnp.float32),
        pltpu.SemaphoreType.DMA,
    ],
    compiler_params=pltpu.CompilerParams(use_tc_tiling_on_sc=False),
)(_sc_body)


def _tc_body(u_ref, lat_ref, att_ref, w_ref, agg_ref, o_ref):
    att = att_ref[...]
    att = att - jnp.max(att, axis=-1, keepdims=True)
    att = jnp.exp(att)
    att = att / jnp.sum(att, axis=-1, keepdims=True)
    disen = lax.dot_general(att, w_ref[...], (((1,), (0,)), ((), ())),
                            preferred_element_type=jnp.float32)
    sc = lax.dot_general(u_ref[...], lat_ref[...], (((1,), (1,)), ((), ())),
                         preferred_element_type=jnp.float32)
    mask = jnp.where(lax.broadcasted_iota(jnp.int32, (1, 8), 1) >= 4,
                     -1e30, 0.0)
    sc = sc + mask
    sc = sc - jnp.max(sc, axis=-1, keepdims=True)
    sc = jnp.exp(sc)
    sc = sc / jnp.sum(sc, axis=-1, keepdims=True)
    coef = 1.0 + lax.dot_general(sc, disen, (((1,), (0,)), ((), ())),
                                 preferred_element_type=jnp.float32)
    o_ref[...] = agg_ref[...] * coef


def _tc_user(user_emb, latent_pad, att_pad, weight, usr_raw):
    blk = 1000
    grid = _N_USERS // blk
    return pl.pallas_call(
        _tc_body,
        grid=(grid,),
        in_specs=[
            pl.BlockSpec((blk, _C), lambda i: (i, 0)),
            pl.BlockSpec((8, _C), lambda i: (0, 0)),
            pl.BlockSpec((8, 8), lambda i: (0, 0)),
            pl.BlockSpec((8, _C), lambda i: (0, 0)),
            pl.BlockSpec((blk, _C), lambda i: (i, 0)),
        ],
        out_specs=pl.BlockSpec((blk, _C), lambda i: (i, 0)),
        out_shape=jax.ShapeDtypeStruct((_N_USERS, _C), jnp.float32),
    )(user_emb, latent_pad, att_pad, weight, usr_raw)


def kernel(entity_emb, user_emb, latent_emb, edge_index, edge_type,
           interact_indices, interact_values, relation_edge_weight,
           weight, disen_weight_att):
    i32 = jnp.int32
    head = edge_index[0]
    tail = edge_index[1]
    pad_e = _EP - _N_EDGES
    tail_pad = jnp.concatenate([tail, jnp.zeros((pad_e,), i32)])
    tail2 = jnp.stack([tail_pad, tail_pad + _N_ENT])
    tp = jnp.concatenate([edge_type - 1, jnp.full((pad_e,), _N_REL - 1, i32)])
    ridx2 = jnp.stack([tp, tp + _N_REL])
    head_pad = jnp.concatenate([head, jnp.full((pad_e,), _N_ENT, i32)])

    rows = interact_indices[0]
    cols = interact_indices[1]
    pad_n = _NP - _NNZ
    cols_pad = jnp.concatenate([cols, jnp.zeros((pad_n,), i32)])
    cols2 = jnp.stack([cols_pad, cols_pad + _N_ENT])
    vals_pad = jnp.concatenate([interact_values,
                                jnp.zeros((pad_n,), jnp.float32)])
    vals_wide = jnp.broadcast_to(vals_pad[:, None], (_NP, 16))
    urows_pad = jnp.concatenate([rows, jnp.full((pad_n,), _N_USERS, i32)])

    ent_cat = jnp.concatenate([entity_emb[:, :_CH], entity_emb[:, _CH:]],
                              axis=0)
    zrow = jnp.zeros((1, _CH), jnp.float32)
    rel_cat = jnp.concatenate(
        [relation_edge_weight[:, :_CH], zrow,
         relation_edge_weight[:, _CH:], zrow], axis=0)

    z64 = jnp.zeros((_STRIPE, _CH), jnp.float32)
    z16 = jnp.zeros((_STRIPE, 16), jnp.float32)
    o16 = jnp.ones((_CHUNK, 16), jnp.float32)

    out_ent, out_usr = _sc_agg(ent_cat, rel_cat, tail2, ridx2, head_pad,
                               cols2, vals_wide, urows_pad, z64, z16, o16)

    entity_agg = jnp.concatenate(
        [out_ent[0, :_N_ENT], out_ent[1, :_N_ENT]], axis=1)
    usr_raw = jnp.concatenate(
        [out_usr[0, :_N_USERS], out_usr[1, :_N_USERS]], axis=1)

    latent_pad = jnp.concatenate(
        [latent_emb, jnp.zeros((4, _C), jnp.float32)], axis=0)
    att_pad = jnp.concatenate(
        [disen_weight_att, jnp.zeros((4, 8), jnp.float32)], axis=0)

    user_agg = _tc_user(user_emb, latent_pad, att_pad, weight, usr_raw)
    return (entity_agg, user_agg)
